# Initial kernel scaffold; baseline (speedup 1.0000x reference)
import jax
import jax.numpy as jnp
from jax.experimental import pallas as pl


def kernel(x, emb_proton, emb_neutron, W1, b1, W2, b2, W3, b3):
    raise NotImplementedError("write your pallas kernel here")


# speedup vs baseline: 2.3936x; 2.3936x over previous
import functools

import jax
import jax.numpy as jnp
from jax import lax
from jax.experimental import pallas as pl
from jax.experimental.pallas import tpu as pltpu
from jax.experimental.pallas import tpu_sc as plsc

B = 16384
H = 128
NC, NS = 2, 16
NW = NC * NS
BPW = B // NW
NCH = BPW // H


def _sc_gather(idx3_p, idx3_n, emb_p, emb_n):
    mesh = plsc.VectorSubcoreMesh(core_axis_name="c", subcore_axis_name="s")

    @functools.partial(
        pl.kernel,
        out_type=(
            jax.ShapeDtypeStruct((B, H), jnp.float32),
            jax.ShapeDtypeStruct((B, H), jnp.float32),
        ),
        mesh=mesh,
        scratch_types=[
            pltpu.VMEM((NCH, H), jnp.int32),
            pltpu.VMEM((NCH, H), jnp.int32),
            pltpu.VMEM((BPW, H), jnp.float32),
            pltpu.SemaphoreType.DMA,
        ],
    )
    def k(ip_hbm, in_hbm, ep_hbm, en_hbm, outp_hbm, outn_hbm,
          idxp_v, idxn_v, rows_v, sem):
        wid = lax.axis_index("s") * NC + lax.axis_index("c")
        base = wid * BPW
        pltpu.sync_copy(ip_hbm.at[wid], idxp_v)
        pltpu.sync_copy(in_hbm.at[wid], idxn_v)
        for tbl, idx_v, out_hbm in ((ep_hbm, idxp_v, outp_hbm),
                                    (en_hbm, idxn_v, outn_hbm)):
            cps = [
                pltpu.make_async_copy(
                    tbl.at[idx_v.at[j]],
                    rows_v.at[pl.ds(j * H, H)],
                    sem,
                )
                for j in range(NCH)
            ]
            for c in cps:
                c.start()
            for c in cps:
                c.wait()
            pltpu.sync_copy(rows_v, out_hbm.at[pl.ds(base, BPW)])

    return k(idx3_p, idx3_n, emb_p, emb_n)


def _tc_mlp(p_rows, n_rows, w1p, w1n, b1, w2, b2, w3r, b3):
    BB = 1024
    grid = (B // BB,)

    def body(p_ref, n_ref, w1p_ref, w1n_ref, b1_ref, w2_ref, b2_ref,
             w3_ref, b3_ref, out_ref):
        p = p_ref[...]
        n = n_ref[...]
        p = p * lax.rsqrt(jnp.maximum(
            jnp.sum(p * p, axis=1, keepdims=True), 1e-24))
        n = n * lax.rsqrt(jnp.maximum(
            jnp.sum(n * n, axis=1, keepdims=True), 1e-24))
        h = jnp.dot(p, w1p_ref[...], preferred_element_type=jnp.float32)
        h = h + jnp.dot(n, w1n_ref[...], preferred_element_type=jnp.float32)
        h = jnp.maximum(h + b1_ref[...], 0.0)
        h = jnp.dot(h, w2_ref[...], preferred_element_type=jnp.float32)
        h = jnp.maximum(h + b2_ref[...], 0.0)
        o = jnp.sum(h * w3_ref[...], axis=1, keepdims=True) + b3_ref[...]
        out_ref[...] = o

    const = lambda i: (0, 0)
    return pl.pallas_call(
        body,
        grid=grid,
        in_specs=[
            pl.BlockSpec((BB, H), lambda i: (i, 0)),
            pl.BlockSpec((BB, H), lambda i: (i, 0)),
            pl.BlockSpec((H, H), const),
            pl.BlockSpec((H, H), const),
            pl.BlockSpec((1, H), const),
            pl.BlockSpec((H, H), const),
            pl.BlockSpec((1, H), const),
            pl.BlockSpec((1, H), const),
            pl.BlockSpec((1, 1), const),
        ],
        out_specs=pl.BlockSpec((BB, 1), lambda i: (i, 0)),
        out_shape=jax.ShapeDtypeStruct((B, 1), jnp.float32),
    )(p_rows, n_rows, w1p, w1n, b1, w2, b2, w3r, b3)


def kernel(x, emb_proton, emb_neutron, W1, b1, W2, b2, W3, b3):
    x = x.astype(jnp.int32)
    idx3_p = x[:, 0].reshape(NW, NCH, H)
    idx3_n = x[:, 1].reshape(NW, NCH, H)
    p_rows, n_rows = _sc_gather(idx3_p, idx3_n, emb_proton, emb_neutron)
    return _tc_mlp(
        p_rows, n_rows,
        W1[:H], W1[H:], b1.reshape(1, H),
        W2, b2.reshape(1, H),
        W3.reshape(1, H), b3.reshape(1, 1),
    )


# speedup vs baseline: 2.4401x; 1.0194x over previous
import functools

import jax
import jax.numpy as jnp
from jax import lax
from jax.experimental import pallas as pl
from jax.experimental.pallas import tpu as pltpu
from jax.experimental.pallas import tpu_sc as plsc

B = 16384
H = 128
NC, NS = 2, 16
NW = NC * NS
NCHUNK = 4
CB = B // NCHUNK
BPW = CB // NW
NCH = BPW // H
BB = 1024


def _sc_gather(idx3_p, idx3_n, emb_p, emb_n):
    mesh = plsc.VectorSubcoreMesh(core_axis_name="c", subcore_axis_name="s")

    @functools.partial(
        pl.kernel,
        out_type=(
            jax.ShapeDtypeStruct((CB, H), jnp.float32),
            jax.ShapeDtypeStruct((CB, H), jnp.float32),
        ),
        mesh=mesh,
        scratch_types=[
            pltpu.VMEM((NCH, H), jnp.int32),
            pltpu.VMEM((NCH, H), jnp.int32),
            pltpu.VMEM((BPW, H), jnp.float32),
            pltpu.SemaphoreType.DMA,
        ],
    )
    def k(ip_hbm, in_hbm, ep_hbm, en_hbm, outp_hbm, outn_hbm,
          idxp_v, idxn_v, rows_v, sem):
        wid = lax.axis_index("s") * NC + lax.axis_index("c")
        base = wid * BPW
        pltpu.sync_copy(ip_hbm.at[wid], idxp_v)
        pltpu.sync_copy(in_hbm.at[wid], idxn_v)
        for tbl, idx_v, out_hbm in ((ep_hbm, idxp_v, outp_hbm),
                                    (en_hbm, idxn_v, outn_hbm)):
            cps = [
                pltpu.make_async_copy(
                    tbl.at[idx_v.at[j]],
                    rows_v.at[pl.ds(j * H, H)],
                    sem,
                )
                for j in range(NCH)
            ]
            for c in cps:
                c.start()
            for c in cps:
                c.wait()
            pltpu.sync_copy(rows_v, out_hbm.at[pl.ds(base, BPW)])

    return k(idx3_p, idx3_n, emb_p, emb_n)


def _tc_mlp(p_rows, n_rows, w1p, w1n, b1, w2, b2, w3r, b3):
    grid = (CB // BB,)

    def body(p_ref, n_ref, w1p_ref, w1n_ref, b1_ref, w2_ref, b2_ref,
             w3_ref, b3_ref, out_ref):
        p = p_ref[...]
        n = n_ref[...]
        p = p * lax.rsqrt(jnp.maximum(
            jnp.sum(p * p, axis=1, keepdims=True), 1e-24))
        n = n * lax.rsqrt(jnp.maximum(
            jnp.sum(n * n, axis=1, keepdims=True), 1e-24))
        h = jnp.dot(p, w1p_ref[...], preferred_element_type=jnp.float32)
        h = h + jnp.dot(n, w1n_ref[...], preferred_element_type=jnp.float32)
        h = jnp.maximum(h + b1_ref[...], 0.0)
        h = jnp.dot(h, w2_ref[...], preferred_element_type=jnp.float32)
        h = jnp.maximum(h + b2_ref[...], 0.0)
        o = lax.dot_general(w3_ref[...], h, (((1,), (1,)), ((), ())),
                            preferred_element_type=jnp.float32)
        out_ref[...] = o + b3_ref[...]

    const = lambda i: (0, 0)
    return pl.pallas_call(
        body,
        grid=grid,
        in_specs=[
            pl.BlockSpec((BB, H), lambda i: (i, 0)),
            pl.BlockSpec((BB, H), lambda i: (i, 0)),
            pl.BlockSpec((H, H), const),
            pl.BlockSpec((H, H), const),
            pl.BlockSpec((1, H), const),
            pl.BlockSpec((H, H), const),
            pl.BlockSpec((1, H), const),
            pl.BlockSpec((1, H), const),
            pl.BlockSpec((1, 1), const),
        ],
        out_specs=pl.BlockSpec((1, BB), lambda i: (0, i)),
        out_shape=jax.ShapeDtypeStruct((1, CB), jnp.float32),
    )(p_rows, n_rows, w1p, w1n, b1, w2, b2, w3r, b3)


def kernel(x, emb_proton, emb_neutron, W1, b1, W2, b2, W3, b3):
    x = x.astype(jnp.int32)
    idx4_p = x[:, 0].reshape(NCHUNK, NW, NCH, H)
    idx4_n = x[:, 1].reshape(NCHUNK, NW, NCH, H)
    w1p, w1n = W1[:H], W1[H:]
    b1r, b2r = b1.reshape(1, H), b2.reshape(1, H)
    w3r, b3r = W3.reshape(1, H), b3.reshape(1, 1)
    outs = []
    for c in range(NCHUNK):
        p_rows, n_rows = _sc_gather(idx4_p[c], idx4_n[c],
                                    emb_proton, emb_neutron)
        outs.append(_tc_mlp(p_rows, n_rows,
                            w1p, w1n, b1r, W2, b2r, w3r, b3r))
    return jnp.concatenate(outs, axis=1).reshape(B, 1)


# speedup vs baseline: 2.7028x; 1.1077x over previous
import functools

import jax
import jax.numpy as jnp
from jax import lax
from jax.experimental import pallas as pl
from jax.experimental.pallas import tpu as pltpu
from jax.experimental.pallas import tpu_sc as plsc

B = 16384
H = 128
NC, NS = 2, 16
NW = NC * NS
NCHUNK = 2
CB = B // NCHUNK
BPW = CB // NW
NCH = BPW // H
BB = 1024


def _sc_gather(idx3_p, idx3_n, emb_p, emb_n):
    mesh = plsc.VectorSubcoreMesh(core_axis_name="c", subcore_axis_name="s")

    @functools.partial(
        pl.kernel,
        out_type=(
            jax.ShapeDtypeStruct((CB, H), jnp.float32),
            jax.ShapeDtypeStruct((CB, H), jnp.float32),
        ),
        mesh=mesh,
        scratch_types=[
            pltpu.VMEM((NCH, H), jnp.int32),
            pltpu.VMEM((NCH, H), jnp.int32),
            pltpu.VMEM((BPW, H), jnp.float32),
            pltpu.SemaphoreType.DMA,
        ],
    )
    def k(ip_hbm, in_hbm, ep_hbm, en_hbm, outp_hbm, outn_hbm,
          idxp_v, idxn_v, rows_v, sem):
        wid = lax.axis_index("s") * NC + lax.axis_index("c")
        base = wid * BPW
        pltpu.sync_copy(ip_hbm.at[wid], idxp_v)
        pltpu.sync_copy(in_hbm.at[wid], idxn_v)
        for tbl, idx_v, out_hbm in ((ep_hbm, idxp_v, outp_hbm),
                                    (en_hbm, idxn_v, outn_hbm)):
            cps = [
                pltpu.make_async_copy(
                    tbl.at[idx_v.at[j]],
                    rows_v.at[pl.ds(j * H, H)],
                    sem,
                )
                for j in range(NCH)
            ]
            for c in cps:
                c.start()
            for c in cps:
                c.wait()
            pltpu.sync_copy(rows_v, out_hbm.at[pl.ds(base, BPW)])

    return k(idx3_p, idx3_n, emb_p, emb_n)


def _tc_mlp(p_rows, n_rows, w1p, w1n, b1, w2, b2, w3r, b3):
    grid = (CB // BB,)

    def body(p_ref, n_ref, w1p_ref, w1n_ref, b1_ref, w2_ref, b2_ref,
             w3_ref, b3_ref, out_ref):
        p = p_ref[...]
        n = n_ref[...]
        p = p * lax.rsqrt(jnp.maximum(
            jnp.sum(p * p, axis=1, keepdims=True), 1e-24))
        n = n * lax.rsqrt(jnp.maximum(
            jnp.sum(n * n, axis=1, keepdims=True), 1e-24))
        h = jnp.dot(p, w1p_ref[...], preferred_element_type=jnp.float32)
        h = h + jnp.dot(n, w1n_ref[...], preferred_element_type=jnp.float32)
        h = jnp.maximum(h + b1_ref[...], 0.0)
        h = jnp.dot(h, w2_ref[...], preferred_element_type=jnp.float32)
        h = jnp.maximum(h + b2_ref[...], 0.0)
        o = lax.dot_general(w3_ref[...], h, (((1,), (1,)), ((), ())),
                            preferred_element_type=jnp.float32)
        out_ref[...] = o + b3_ref[...]

    const = lambda i: (0, 0)
    return pl.pallas_call(
        body,
        grid=grid,
        in_specs=[
            pl.BlockSpec((BB, H), lambda i: (i, 0)),
            pl.BlockSpec((BB, H), lambda i: (i, 0)),
            pl.BlockSpec((H, H), const),
            pl.BlockSpec((H, H), const),
            pl.BlockSpec((1, H), const),
            pl.BlockSpec((H, H), const),
            pl.BlockSpec((1, H), const),
            pl.BlockSpec((1, H), const),
            pl.BlockSpec((1, 1), const),
        ],
        out_specs=pl.BlockSpec((1, BB), lambda i: (0, i)),
        out_shape=jax.ShapeDtypeStruct((1, CB), jnp.float32),
    )(p_rows, n_rows, w1p, w1n, b1, w2, b2, w3r, b3)


def kernel(x, emb_proton, emb_neutron, W1, b1, W2, b2, W3, b3):
    x = x.astype(jnp.int32)
    idx4_p = x[:, 0].reshape(NCHUNK, NW, NCH, H)
    idx4_n = x[:, 1].reshape(NCHUNK, NW, NCH, H)
    w1p, w1n = W1[:H], W1[H:]
    b1r, b2r = b1.reshape(1, H), b2.reshape(1, H)
    w3r, b3r = W3.reshape(1, H), b3.reshape(1, 1)
    outs = []
    for c in range(NCHUNK):
        p_rows, n_rows = _sc_gather(idx4_p[c], idx4_n[c],
                                    emb_proton, emb_neutron)
        outs.append(_tc_mlp(p_rows, n_rows,
                            w1p, w1n, b1r, W2, b2r, w3r, b3r))
    return jnp.concatenate(outs, axis=1).reshape(B, 1)
